# baseline (speedup 1.0000x reference)
import functools

import jax
import jax.numpy as jnp
from jax import lax
from jax.experimental import pallas as pl
from jax.experimental.pallas import tpu as pltpu
from jax.experimental.pallas import tpu_sc as plsc

NC = 2
NS = 16
L = 16


def _sc_mesh():
    return plsc.VectorSubcoreMesh(core_axis_name="c", subcore_axis_name="s")


G_CH = 8


def _hist_body(n_pad, groups, cols_hbm, out_hbm, col_v, hist, blk, acc, stage):
    c = lax.axis_index("c")
    s = lax.axis_index("s")
    w = c * NS + s
    npt = n_pad // NS

    zero16 = jnp.zeros((L,), jnp.float32)
    ones16 = jnp.ones((L,), jnp.float32)

    def zbody(i, carry):
        hist[pl.ds(i * L, L)] = zero16
        return carry

    lax.fori_loop(0, n_pad // L, zbody, 0)

    def cbody(g, carry):
        pltpu.sync_copy(cols_hbm.at[w, pl.ds(g * G_CH, G_CH)], col_v)
        for j in range(G_CH):
            for k in range(128 // L):
                idx = col_v[j, pl.ds(k * L, L)]
                plsc.addupdate_scatter(hist, [idx], ones16)
        return carry

    lax.fori_loop(0, groups, cbody, 0)

    for b in range(NS):
        pltpu.sync_copy(hist.at[pl.ds(b * npt, npt)], stage.at[b, s])
    plsc.subcore_barrier()

    pltpu.sync_copy(stage.at[s], blk)

    def rbody(g, carry):
        v = blk[0, pl.ds(g * L, L)]
        for r in range(1, NS):
            v = v + blk[r, pl.ds(g * L, L)]
        acc[pl.ds(g * L, L)] = v
        return carry

    lax.fori_loop(0, npt // L, rbody, 0)
    pltpu.sync_copy(acc, out_hbm.at[pl.ds(c * n_pad + s * npt, npt)])


def _make_hist(n_pad, groups):
    npt = n_pad // NS
    return pl.kernel(
        functools.partial(_hist_body, n_pad, groups),
        out_type=jax.ShapeDtypeStruct((NC * n_pad,), jnp.float32),
        mesh=_sc_mesh(),
        scratch_types=[
            pltpu.VMEM((G_CH, 128), jnp.int32),
            pltpu.VMEM((n_pad,), jnp.float32),
            pltpu.VMEM((NS, npt), jnp.float32),
            pltpu.VMEM((npt,), jnp.float32),
            pltpu.VMEM_SHARED((NS, NS, npt), jnp.float32),
        ],
        compiler_params=pltpu.CompilerParams(needs_layout_passes=False),
    )


def _prop_body(n_pad, groups, d_in, xs_hbm, rows_hbm, cols_hbm, zrows_hbm,
               out_hbm, row_v, col_v, gbuf, acc, sem0, sem1, ssem0, ssem1):
    c = lax.axis_index("c")
    s = lax.axis_index("s")
    w = c * NS + s
    npt = n_pad // NS
    sems = (sem0, sem1)
    ssems = (ssem0, ssem1)

    pltpu.sync_copy(zrows_hbm.at[pl.ds(s * npt, npt)],
                    acc.at[pl.ds(s * npt, npt)])
    plsc.subcore_barrier()

    def body(g, carry):
        pltpu.sync_copy(rows_hbm.at[w, pl.ds(g * G_CH, G_CH)], row_v)
        pltpu.sync_copy(cols_hbm.at[w, pl.ds(g * G_CH, G_CH)], col_v)
        pltpu.async_copy(xs_hbm.at[row_v.at[0]], gbuf.at[0], sems[0])
        for j in range(G_CH):
            p = j % 2
            if j + 1 < G_CH:
                if j >= 1:
                    pltpu.make_async_copy(gbuf.at[1 - p],
                                          acc.at[col_v.at[j - 1]],
                                          ssems[1 - p]).wait()
                pltpu.async_copy(xs_hbm.at[row_v.at[j + 1]],
                                 gbuf.at[1 - p], sems[1 - p])
            pltpu.make_async_copy(xs_hbm.at[row_v.at[j]],
                                  gbuf.at[p], sems[p]).wait()
            pltpu.async_copy(gbuf.at[p], acc.at[col_v.at[j]], ssems[p],
                             add=True)
        pltpu.make_async_copy(gbuf.at[0], acc.at[col_v.at[G_CH - 2]],
                              ssems[0]).wait()
        pltpu.make_async_copy(gbuf.at[1], acc.at[col_v.at[G_CH - 1]],
                              ssems[1]).wait()
        return carry

    lax.fori_loop(0, groups, body, 0)
    plsc.subcore_barrier()
    pltpu.sync_copy(acc.at[pl.ds(s * npt, npt)],
                    out_hbm.at[pl.ds(c * n_pad + s * npt, npt)])


def _make_prop(n_pad, groups, d_in):
    return pl.kernel(
        functools.partial(_prop_body, n_pad, groups, d_in),
        out_type=jax.ShapeDtypeStruct((2 * n_pad, d_in), jnp.float32),
        mesh=_sc_mesh(),
        scratch_types=[
            pltpu.VMEM((G_CH, 128), jnp.int32),
            pltpu.VMEM((G_CH, 128), jnp.int32),
            pltpu.VMEM((2, 128, d_in), jnp.float32),
            pltpu.VMEM_SHARED((n_pad, d_in), jnp.float32),
            pltpu.SemaphoreType.DMA,
            pltpu.SemaphoreType.DMA,
            pltpu.SemaphoreType.DMA,
            pltpu.SemaphoreType.DMA,
        ],
    )


def _pair_body(pchunks, d_h, h_hbm, idx_hbm, out_hbm, idx_v, rbuf, sem):
    c = lax.axis_index("c")
    s = lax.axis_index("s")
    w = c * NS + s
    for j in range(pchunks):
        pltpu.async_copy(h_hbm.at[idx_v.at[j]], rbuf, sem).wait()
        pltpu.sync_copy(rbuf, out_hbm.at[pl.ds((w * pchunks + j) * 128, 128)])


def _pair_prelude(pchunks, h_hbm, idx_hbm, out_hbm, idx_v, rbuf, sem):
    c = lax.axis_index("c")
    s = lax.axis_index("s")
    w = c * NS + s
    pltpu.sync_copy(idx_hbm.at[w], idx_v)


def _full_pair_body(pchunks, d_h, h_hbm, idx_hbm, out_hbm, idx_v, rbuf, sem):
    _pair_prelude(pchunks, h_hbm, idx_hbm, out_hbm, idx_v, rbuf, sem)
    _pair_body(pchunks, d_h, h_hbm, idx_hbm, out_hbm, idx_v, rbuf, sem)


def _make_pair(total_rows, pchunks, d_h):
    return pl.kernel(
        functools.partial(_full_pair_body, pchunks, d_h),
        out_type=jax.ShapeDtypeStruct((total_rows, d_h), jnp.float32),
        mesh=_sc_mesh(),
        scratch_types=[
            pltpu.VMEM((pchunks, 128), jnp.int32),
            pltpu.VMEM((128, d_h), jnp.float32),
            pltpu.SemaphoreType.DMA,
        ],
    )


def _prescale_body(cnt_ref, x_ref, xs_ref, dinv_ref):
    deg = cnt_ref[...] + 1.0
    dinv = lax.rsqrt(deg)
    xs_ref[...] = x_ref[...] * dinv
    dinv_ref[...] = dinv


def _tc_prescale(cnt2, x_pad, n_pad, d_in):
    nblk = 4
    bn = n_pad // nblk
    grid = 2 * nblk
    return pl.pallas_call(
        _prescale_body,
        grid=(grid,),
        in_specs=[
            pl.BlockSpec((bn, 1), lambda b: (b, 0)),
            pl.BlockSpec((bn, d_in), lambda b: (b % nblk, 0)),
        ],
        out_specs=[
            pl.BlockSpec((bn, d_in), lambda b: (b, 0)),
            pl.BlockSpec((bn, 1), lambda b: (b, 0)),
        ],
        out_shape=[
            jax.ShapeDtypeStruct((2 * n_pad, d_in), jnp.float32),
            jax.ShapeDtypeStruct((2 * n_pad, 1), jnp.float32),
        ],
    )(cnt2, x_pad)


def _encode_body(raw1_ref, raw2_ref, x_ref, dinv1_ref, dinv2_ref, wt_ref,
                 b_ref, h_ref):
    x = x_ref[...]
    wt = wt_ref[...]
    b = b_ref[...]

    def enc(raw, dinv):
        prop = dinv * raw + (dinv * dinv) * x
        z = jnp.dot(prop, wt, preferred_element_type=jnp.float32) + b
        return jnp.where(z >= 0.0, z, 0.1 * z)

    y1 = enc(raw1_ref[...], dinv1_ref[...])
    y2 = enc(raw2_ref[...], dinv2_ref[...])
    h_ref[...] = y2 - y1


def _tc_encode(raw, x_pad, dinv, wt, b2, n_pad, d_in, d_h):
    nblk = 4
    bn = n_pad // nblk
    return pl.pallas_call(
        _encode_body,
        grid=(nblk,),
        in_specs=[
            pl.BlockSpec((bn, d_in), lambda b: (b, 0)),
            pl.BlockSpec((bn, d_in), lambda b: (b + nblk, 0)),
            pl.BlockSpec((bn, d_in), lambda b: (b, 0)),
            pl.BlockSpec((bn, 1), lambda b: (b, 0)),
            pl.BlockSpec((bn, 1), lambda b: (b + nblk, 0)),
            pl.BlockSpec((d_in, d_h), lambda b: (0, 0)),
            pl.BlockSpec((1, d_h), lambda b: (0, 0)),
        ],
        out_specs=pl.BlockSpec((bn, d_h), lambda b: (b, 0)),
        out_shape=jax.ShapeDtypeStruct((n_pad, d_h), jnp.float32),
    )(raw, raw, x_pad, dinv, dinv, wt, b2)


def _decode_body(l_ref, r_ref, wd_ref, bd_ref, out_ref):
    d = l_ref[...] - r_ref[...]
    q = -(d * d)
    m = jnp.max(q, axis=1, keepdims=True)
    e = jnp.exp(q - m)
    p = e / jnp.sum(e, axis=1, keepdims=True)
    out_ref[...] = (
        jnp.dot(p, wd_ref[...], preferred_element_type=jnp.float32)
        + bd_ref[...]
    )


def _tc_decode(pairs, wd_t, bd2, m, d_h, d_out):
    nblk = 4
    bm = m // nblk
    return pl.pallas_call(
        _decode_body,
        grid=(nblk,),
        in_specs=[
            pl.BlockSpec((bm, d_h), lambda b: (b, 0)),
            pl.BlockSpec((bm, d_h), lambda b: (b + nblk, 0)),
            pl.BlockSpec((d_h, d_out), lambda b: (0, 0)),
            pl.BlockSpec((1, d_out), lambda b: (0, 0)),
        ],
        out_specs=pl.BlockSpec((bm, d_out), lambda b: (b, 0)),
        out_shape=jax.ShapeDtypeStruct((m, d_out), jnp.float32),
    )(pairs, pairs, wd_t, bd2)


def kernel(x, edge_index1, edge_index2, modified, W_enc, b_enc, W_dec, b_dec):
    n, d_in = x.shape
    e = edge_index1.shape[1]
    m = modified.shape[1]
    d_h, _ = W_enc.shape
    d_out = W_dec.shape[0]

    n_pad = -(-n // 256) * 256
    if n_pad == n:
        n_pad += 256

    ept = -(-e // NS)
    chunks = -(-(-(-ept // 128)) // G_CH) * G_CH
    groups = chunks // G_CH
    e_pad = NS * chunks * 128

    rows_list, cols_list = [], []
    for c, ei in enumerate((edge_index1, edge_index2)):
        r = jnp.concatenate(
            [ei[0], jnp.zeros((e_pad - e,), jnp.int32)]) + c * n_pad
        cl = jnp.concatenate(
            [ei[1], jnp.full((e_pad - e,), n, jnp.int32)])
        rows_list.append(r.reshape(NS, chunks, 128))
        cols_list.append(cl.reshape(NS, chunks, 128))
    rows_idx = jnp.concatenate(rows_list, axis=0)
    cols_idx = jnp.concatenate(cols_list, axis=0)

    x_pad = jnp.pad(x, ((0, n_pad - n), (0, 0)))

    cnt = _make_hist(n_pad, groups)(cols_idx)
    xs, dinv = _tc_prescale(cnt.reshape(2 * n_pad, 1), x_pad, n_pad, d_in)

    zrows = jnp.zeros((n_pad, d_in), jnp.float32)
    raw = _make_prop(n_pad, groups, d_in)(xs, rows_idx, cols_idx, zrows)

    h = _tc_encode(raw, x_pad, dinv, W_enc.T,
                   b_enc.reshape(1, d_h), n_pad, d_in, d_h)

    total_rows = 2 * m
    pchunks = total_rows // (NC * NS * 128)
    pairs_idx = jnp.concatenate([modified[0], modified[1]]).reshape(
        NC * NS, pchunks, 128)
    pairs = _make_pair(total_rows, pchunks, d_h)(h, pairs_idx)

    return _tc_decode(pairs, W_dec.T, b_dec.reshape(1, d_out), m, d_h, d_out)


# speedup vs baseline: 1.0273x; 1.0273x over previous
import functools

import jax
import jax.numpy as jnp
from jax import lax
from jax.experimental import pallas as pl
from jax.experimental.pallas import tpu as pltpu
from jax.experimental.pallas import tpu_sc as plsc

NC = 2
NS = 16
L = 16


def _sc_mesh():
    return plsc.VectorSubcoreMesh(core_axis_name="c", subcore_axis_name="s")


G_CH = 8


def _hist_body(n_pad, groups, cols_hbm, out_hbm, col_v, hist, blk, acc, stage):
    c = lax.axis_index("c")
    s = lax.axis_index("s")
    w = c * NS + s
    npt = n_pad // NS

    zero16 = jnp.zeros((L,), jnp.float32)
    ones16 = jnp.ones((L,), jnp.float32)

    def zbody(i, carry):
        hist[pl.ds(i * L, L)] = zero16
        return carry

    lax.fori_loop(0, n_pad // L, zbody, 0)

    def cbody(g, carry):
        pltpu.sync_copy(cols_hbm.at[w, pl.ds(g * G_CH, G_CH)], col_v)
        for j in range(G_CH):
            for k in range(128 // L):
                idx = col_v[j, pl.ds(k * L, L)]
                plsc.addupdate_scatter(hist, [idx], ones16)
        return carry

    lax.fori_loop(0, groups, cbody, 0)

    for b in range(NS):
        pltpu.sync_copy(hist.at[pl.ds(b * npt, npt)], stage.at[b, s])
    plsc.subcore_barrier()

    pltpu.sync_copy(stage.at[s], blk)

    def rbody(g, carry):
        v = blk[0, pl.ds(g * L, L)]
        for r in range(1, NS):
            v = v + blk[r, pl.ds(g * L, L)]
        acc[pl.ds(g * L, L)] = v
        return carry

    lax.fori_loop(0, npt // L, rbody, 0)
    pltpu.sync_copy(acc, out_hbm.at[pl.ds(c * n_pad + s * npt, npt)])


def _make_hist(n_pad, groups):
    npt = n_pad // NS
    return pl.kernel(
        functools.partial(_hist_body, n_pad, groups),
        out_type=jax.ShapeDtypeStruct((NC * n_pad,), jnp.float32),
        mesh=_sc_mesh(),
        scratch_types=[
            pltpu.VMEM((G_CH, 128), jnp.int32),
            pltpu.VMEM((n_pad,), jnp.float32),
            pltpu.VMEM((NS, npt), jnp.float32),
            pltpu.VMEM((npt,), jnp.float32),
            pltpu.VMEM_SHARED((NS, NS, npt), jnp.float32),
        ],
        compiler_params=pltpu.CompilerParams(needs_layout_passes=False),
    )


def _prop_body(n_pad, groups, d_in, xs_hbm, rows_hbm, cols_hbm, zrows_hbm,
               out_hbm, row_v0, col_v0, row_v1, col_v1, gbuf, acc,
               sem0, sem1, ssem0, ssem1, isem):
    c = lax.axis_index("c")
    s = lax.axis_index("s")
    w = c * NS + s
    npt = n_pad // NS
    sems = (sem0, sem1)
    ssems = (ssem0, ssem1)
    ibufs = ((row_v0, col_v0), (row_v1, col_v1))

    pltpu.sync_copy(zrows_hbm.at[pl.ds(s * npt, npt)],
                    acc.at[pl.ds(s * npt, npt)])
    plsc.subcore_barrier()

    pltpu.sync_copy(rows_hbm.at[w, pl.ds(0, G_CH)], row_v0)
    pltpu.sync_copy(cols_hbm.at[w, pl.ds(0, G_CH)], col_v0)

    def half(g, ip):
        row_v, col_v = ibufs[ip]
        nrow_v, ncol_v = ibufs[1 - ip]

        @pl.when(g + 1 < groups)
        def _():
            pltpu.async_copy(rows_hbm.at[w, pl.ds((g + 1) * G_CH, G_CH)],
                             nrow_v, isem)
            pltpu.async_copy(cols_hbm.at[w, pl.ds((g + 1) * G_CH, G_CH)],
                             ncol_v, isem)

        pltpu.async_copy(xs_hbm.at[row_v.at[0]], gbuf.at[0], sems[0])
        for j in range(G_CH):
            p = j % 2
            if j + 1 < G_CH:
                if j >= 1:
                    pltpu.make_async_copy(gbuf.at[1 - p],
                                          acc.at[col_v.at[j - 1]],
                                          ssems[1 - p]).wait()
                pltpu.async_copy(xs_hbm.at[row_v.at[j + 1]],
                                 gbuf.at[1 - p], sems[1 - p])
            pltpu.make_async_copy(xs_hbm.at[row_v.at[j]],
                                  gbuf.at[p], sems[p]).wait()
            pltpu.async_copy(gbuf.at[p], acc.at[col_v.at[j]], ssems[p],
                             add=True)
        pltpu.make_async_copy(gbuf.at[0], acc.at[col_v.at[G_CH - 2]],
                              ssems[0]).wait()
        pltpu.make_async_copy(gbuf.at[1], acc.at[col_v.at[G_CH - 1]],
                              ssems[1]).wait()

        @pl.when(g + 1 < groups)
        def _():
            pltpu.make_async_copy(
                rows_hbm.at[w, pl.ds((g + 1) * G_CH, G_CH)], nrow_v,
                isem).wait()
            pltpu.make_async_copy(
                cols_hbm.at[w, pl.ds((g + 1) * G_CH, G_CH)], ncol_v,
                isem).wait()

    def body(gg, carry):
        half(2 * gg, 0)
        half(2 * gg + 1, 1)
        return carry

    lax.fori_loop(0, groups // 2, body, 0)
    plsc.subcore_barrier()
    pltpu.sync_copy(acc.at[pl.ds(s * npt, npt)],
                    out_hbm.at[pl.ds(c * n_pad + s * npt, npt)])


def _make_prop(n_pad, groups, d_in):
    return pl.kernel(
        functools.partial(_prop_body, n_pad, groups, d_in),
        out_type=jax.ShapeDtypeStruct((2 * n_pad, d_in), jnp.float32),
        mesh=_sc_mesh(),
        scratch_types=[
            pltpu.VMEM((G_CH, 128), jnp.int32),
            pltpu.VMEM((G_CH, 128), jnp.int32),
            pltpu.VMEM((G_CH, 128), jnp.int32),
            pltpu.VMEM((G_CH, 128), jnp.int32),
            pltpu.VMEM((2, 128, d_in), jnp.float32),
            pltpu.VMEM_SHARED((n_pad, d_in), jnp.float32),
            pltpu.SemaphoreType.DMA,
            pltpu.SemaphoreType.DMA,
            pltpu.SemaphoreType.DMA,
            pltpu.SemaphoreType.DMA,
            pltpu.SemaphoreType.DMA,
        ],
    )


def _pair_body(pchunks, d_h, h_hbm, idx_hbm, out_hbm, idx_v, rbuf, sem):
    c = lax.axis_index("c")
    s = lax.axis_index("s")
    w = c * NS + s
    for j in range(pchunks):
        pltpu.async_copy(h_hbm.at[idx_v.at[j]], rbuf, sem).wait()
        pltpu.sync_copy(rbuf, out_hbm.at[pl.ds((w * pchunks + j) * 128, 128)])


def _pair_prelude(pchunks, h_hbm, idx_hbm, out_hbm, idx_v, rbuf, sem):
    c = lax.axis_index("c")
    s = lax.axis_index("s")
    w = c * NS + s
    pltpu.sync_copy(idx_hbm.at[w], idx_v)


def _full_pair_body(pchunks, d_h, h_hbm, idx_hbm, out_hbm, idx_v, rbuf, sem):
    _pair_prelude(pchunks, h_hbm, idx_hbm, out_hbm, idx_v, rbuf, sem)
    _pair_body(pchunks, d_h, h_hbm, idx_hbm, out_hbm, idx_v, rbuf, sem)


def _make_pair(total_rows, pchunks, d_h):
    return pl.kernel(
        functools.partial(_full_pair_body, pchunks, d_h),
        out_type=jax.ShapeDtypeStruct((total_rows, d_h), jnp.float32),
        mesh=_sc_mesh(),
        scratch_types=[
            pltpu.VMEM((pchunks, 128), jnp.int32),
            pltpu.VMEM((128, d_h), jnp.float32),
            pltpu.SemaphoreType.DMA,
        ],
    )


def _prescale_body(cnt_ref, x_ref, xs_ref, dinv_ref):
    deg = cnt_ref[...] + 1.0
    dinv = lax.rsqrt(deg)
    xs_ref[...] = x_ref[...] * dinv
    dinv_ref[...] = dinv


def _tc_prescale(cnt2, x_pad, n_pad, d_in):
    nblk = 4
    bn = n_pad // nblk
    grid = 2 * nblk
    return pl.pallas_call(
        _prescale_body,
        grid=(grid,),
        in_specs=[
            pl.BlockSpec((bn, 1), lambda b: (b, 0)),
            pl.BlockSpec((bn, d_in), lambda b: (b % nblk, 0)),
        ],
        out_specs=[
            pl.BlockSpec((bn, d_in), lambda b: (b, 0)),
            pl.BlockSpec((bn, 1), lambda b: (b, 0)),
        ],
        out_shape=[
            jax.ShapeDtypeStruct((2 * n_pad, d_in), jnp.float32),
            jax.ShapeDtypeStruct((2 * n_pad, 1), jnp.float32),
        ],
    )(cnt2, x_pad)


def _encode_body(raw1_ref, raw2_ref, x_ref, dinv1_ref, dinv2_ref, wt_ref,
                 b_ref, h_ref):
    x = x_ref[...]
    wt = wt_ref[...]
    b = b_ref[...]

    def enc(raw, dinv):
        prop = dinv * raw + (dinv * dinv) * x
        z = jnp.dot(prop, wt, preferred_element_type=jnp.float32) + b
        return jnp.where(z >= 0.0, z, 0.1 * z)

    y1 = enc(raw1_ref[...], dinv1_ref[...])
    y2 = enc(raw2_ref[...], dinv2_ref[...])
    h_ref[...] = y2 - y1


def _tc_encode(raw, x_pad, dinv, wt, b2, n_pad, d_in, d_h):
    nblk = 4
    bn = n_pad // nblk
    return pl.pallas_call(
        _encode_body,
        grid=(nblk,),
        in_specs=[
            pl.BlockSpec((bn, d_in), lambda b: (b, 0)),
            pl.BlockSpec((bn, d_in), lambda b: (b + nblk, 0)),
            pl.BlockSpec((bn, d_in), lambda b: (b, 0)),
            pl.BlockSpec((bn, 1), lambda b: (b, 0)),
            pl.BlockSpec((bn, 1), lambda b: (b + nblk, 0)),
            pl.BlockSpec((d_in, d_h), lambda b: (0, 0)),
            pl.BlockSpec((1, d_h), lambda b: (0, 0)),
        ],
        out_specs=pl.BlockSpec((bn, d_h), lambda b: (b, 0)),
        out_shape=jax.ShapeDtypeStruct((n_pad, d_h), jnp.float32),
    )(raw, raw, x_pad, dinv, dinv, wt, b2)


def _decode_body(l_ref, r_ref, wd_ref, bd_ref, out_ref):
    d = l_ref[...] - r_ref[...]
    q = -(d * d)
    m = jnp.max(q, axis=1, keepdims=True)
    e = jnp.exp(q - m)
    p = e / jnp.sum(e, axis=1, keepdims=True)
    out_ref[...] = (
        jnp.dot(p, wd_ref[...], preferred_element_type=jnp.float32)
        + bd_ref[...]
    )


def _tc_decode(pairs, wd_t, bd2, m, d_h, d_out):
    nblk = 4
    bm = m // nblk
    return pl.pallas_call(
        _decode_body,
        grid=(nblk,),
        in_specs=[
            pl.BlockSpec((bm, d_h), lambda b: (b, 0)),
            pl.BlockSpec((bm, d_h), lambda b: (b + nblk, 0)),
            pl.BlockSpec((d_h, d_out), lambda b: (0, 0)),
            pl.BlockSpec((1, d_out), lambda b: (0, 0)),
        ],
        out_specs=pl.BlockSpec((bm, d_out), lambda b: (b, 0)),
        out_shape=jax.ShapeDtypeStruct((m, d_out), jnp.float32),
    )(pairs, pairs, wd_t, bd2)


def kernel(x, edge_index1, edge_index2, modified, W_enc, b_enc, W_dec, b_dec):
    n, d_in = x.shape
    e = edge_index1.shape[1]
    m = modified.shape[1]
    d_h, _ = W_enc.shape
    d_out = W_dec.shape[0]

    n_pad = -(-n // 256) * 256
    if n_pad == n:
        n_pad += 256

    ept = -(-e // NS)
    chunks = -(-(-(-ept // 128)) // (2 * G_CH)) * 2 * G_CH
    groups = chunks // G_CH
    e_pad = NS * chunks * 128

    rows_list, cols_list = [], []
    for c, ei in enumerate((edge_index1, edge_index2)):
        r = jnp.concatenate(
            [ei[0], jnp.zeros((e_pad - e,), jnp.int32)]) + c * n_pad
        cl = jnp.concatenate(
            [ei[1], jnp.full((e_pad - e,), n, jnp.int32)])
        rows_list.append(r.reshape(NS, chunks, 128))
        cols_list.append(cl.reshape(NS, chunks, 128))
    rows_idx = jnp.concatenate(rows_list, axis=0)
    cols_idx = jnp.concatenate(cols_list, axis=0)

    x_pad = jnp.pad(x, ((0, n_pad - n), (0, 0)))

    cnt = _make_hist(n_pad, groups)(cols_idx)
    xs, dinv = _tc_prescale(cnt.reshape(2 * n_pad, 1), x_pad, n_pad, d_in)

    zrows = jnp.zeros((n_pad, d_in), jnp.float32)
    raw = _make_prop(n_pad, groups, d_in)(xs, rows_idx, cols_idx, zrows)

    h = _tc_encode(raw, x_pad, dinv, W_enc.T,
                   b_enc.reshape(1, d_h), n_pad, d_in, d_h)

    total_rows = 2 * m
    pchunks = total_rows // (NC * NS * 128)
    pairs_idx = jnp.concatenate([modified[0], modified[1]]).reshape(
        NC * NS, pchunks, 128)
    pairs = _make_pair(total_rows, pchunks, d_h)(h, pairs_idx)

    return _tc_decode(pairs, W_dec.T, b_dec.reshape(1, d_out), m, d_h, d_out)
